# baseline (speedup 1.0000x reference)
import jax
import jax.numpy as jnp
from jax.experimental import pallas as pl
from jax.experimental.pallas import tpu as pltpu

D_IN = 768
D_HIDDEN = 8192
K = 64
N_TOK = 2048

TB = 128
CHUNK = 256
N_CHUNK = D_HIDDEN // CHUNK
TB_DEC = 256


def _pipeline_phase(x_blk, be_ref, w_vmem, rd_scr, wr_scr, o_ref):
    lo0 = jnp.zeros((TB, 1), jnp.int32)
    hi0 = jnp.full((TB, 1), jnp.int32(0x7F800000))

    def body(c, carry):
        lo, hi = carry
        wc = w_vmem[pl.ds(c * CHUNK, CHUNK), :]
        enc = jax.lax.dot_general(
            x_blk, wc, (((1,), (1,)), ((), ())),
            preferred_element_type=jnp.float32)
        enc = enc + be_ref[:, pl.ds(c * CHUNK, CHUNK)]
        wr_scr[:, pl.ds(c * CHUNK, CHUNK)] = jnp.maximum(enc, 0.0)
        si = jax.lax.bitcast_convert_type(rd_scr[...], jnp.int32)
        mid = lo + (hi - lo) // 2
        neg = jax.lax.shift_right_arithmetic(si - mid, 31)
        cnt = jnp.sum(neg, axis=1, keepdims=True)
        pred = cnt >= (K - D_HIDDEN)
        lo = jnp.where(pred, mid, lo)
        hi = jnp.where(pred, hi, mid)
        return lo, hi

    lo, _ = jax.lax.fori_loop(0, N_CHUNK, body, (lo0, hi0))

    s_prev = rd_scr[...]
    si_prev = jax.lax.bitcast_convert_type(s_prev, jnp.int32)
    o_ref[...] = jnp.where(si_prev >= lo, s_prev, 0.0)


def _encode_topk_kernel(x_ref, w_hbm, be_ref, o_ref, w_vmem, s_scrA, s_scrB,
                        sem):
    i = pl.program_id(0)

    @pl.when(i == 0)
    def _():
        cp = pltpu.make_async_copy(w_hbm, w_vmem, sem)
        cp.start()
        cp.wait()

    x_blk = x_ref[...]

    @pl.when(i % 2 == 0)
    def _():
        _pipeline_phase(x_blk, be_ref, w_vmem, s_scrB, s_scrA, o_ref)

    @pl.when(i % 2 == 1)
    def _():
        _pipeline_phase(x_blk, be_ref, w_vmem, s_scrA, s_scrB, o_ref)


def _decode_kernel(s_ref, w_hbm, bd_ref, o_ref, w_vmem, sem):
    @pl.when(pl.program_id(0) == 0)
    def _():
        cp = pltpu.make_async_copy(w_hbm, w_vmem, sem)
        cp.start()
        cp.wait()

    o_ref[...] = (
        jnp.dot(s_ref[...], w_vmem[...], preferred_element_type=jnp.float32)
        + bd_ref[...]
    )


@jax.jit
def kernel(x, W_enc, b_enc, b_dec):
    n = x.shape[0]
    nblk = n // TB

    sparse = pl.pallas_call(
        _encode_topk_kernel,
        grid=(nblk + 1,),
        in_specs=[
            pl.BlockSpec((TB, D_IN), lambda i: (jnp.minimum(i, nblk - 1), 0)),
            pl.BlockSpec(memory_space=pl.ANY),
            pl.BlockSpec((1, D_HIDDEN), lambda i: (0, 0)),
        ],
        out_specs=pl.BlockSpec(
            (TB, D_HIDDEN), lambda i: (jnp.maximum(i - 1, 0), 0)),
        out_shape=jax.ShapeDtypeStruct((n, D_HIDDEN), jnp.float32),
        scratch_shapes=[
            pltpu.VMEM((D_HIDDEN, D_IN), jnp.float32),
            pltpu.VMEM((TB, D_HIDDEN), jnp.float32),
            pltpu.VMEM((TB, D_HIDDEN), jnp.float32),
            pltpu.SemaphoreType.DMA,
        ],
    )(x, W_enc, b_enc.reshape(1, D_HIDDEN))

    out = pl.pallas_call(
        _decode_kernel,
        grid=(n // TB_DEC,),
        in_specs=[
            pl.BlockSpec((TB_DEC, D_HIDDEN), lambda i: (i, 0)),
            pl.BlockSpec(memory_space=pl.ANY),
            pl.BlockSpec((1, D_IN), lambda i: (0, 0)),
        ],
        out_specs=pl.BlockSpec((TB_DEC, D_IN), lambda i: (i, 0)),
        out_shape=jax.ShapeDtypeStruct((n, D_IN), jnp.float32),
        scratch_shapes=[
            pltpu.VMEM((D_HIDDEN, D_IN), jnp.float32),
            pltpu.SemaphoreType.DMA,
        ],
    )(sparse, W_enc, b_dec.reshape(1, D_IN))

    return out


# speedup vs baseline: 1.1342x; 1.1342x over previous
import jax
import jax.numpy as jnp
from jax.experimental import pallas as pl
from jax.experimental.pallas import tpu as pltpu

D_IN = 768
D_HIDDEN = 8192
K = 64
N_TOK = 2048

TB = 128
TB_DEC = 256


def _encode_topk_kernel(x_ref, w_hbm, be_ref, o_ref, w_vmem, sem):
    i = pl.program_id(0)

    @pl.when(i == 0)
    def _():
        cp = pltpu.make_async_copy(w_hbm, w_vmem, sem)
        cp.start()
        cp.wait()

    enc = jax.lax.dot_general(
        x_ref[...], w_vmem[...], (((1,), (1,)), ((), ())),
        preferred_element_type=jnp.float32)
    s = jnp.maximum(enc + be_ref[...], 0.0)
    si = jax.lax.bitcast_convert_type(s, jnp.int32)

    lo0 = jnp.zeros((TB, 1), jnp.int32)
    hi0 = jnp.full((TB, 1), jnp.int32(0x7F800000))
    thr0 = jnp.zeros((TB, 1), jnp.int32)
    done0 = jnp.zeros((TB, 1), jnp.int32)

    def cond(carry):
        it, _, _, _, alldone = carry
        return jnp.logical_and(it < 31, jnp.logical_not(alldone))

    def body(carry):
        it, lo, hi, thr_done, _ = carry
        thr, done = thr_done
        mid = lo + (hi - lo) // 2
        neg = jax.lax.shift_right_arithmetic(si - mid, 31)
        cnt = jnp.sum(neg, axis=1, keepdims=True) + D_HIDDEN
        pred = cnt >= K
        lo = jnp.where(pred, mid, lo)
        hi = jnp.where(pred, hi, mid)
        newly = jnp.logical_and(cnt == K, done == 0)
        thr = jnp.where(newly, mid, thr)
        done = jnp.where(newly, 1, done)
        alldone = jnp.min(done) == 1
        return it + 1, lo, hi, (thr, done), alldone

    _, lo, _, (thr, done), _ = jax.lax.while_loop(
        cond, body, (0, lo0, hi0, (thr0, done0), False))
    thr = jnp.where(done == 1, thr, lo)

    o_ref[...] = jnp.where(si >= thr, s, 0.0)


def _decode_kernel(s_ref, w_hbm, bd_ref, o_ref, w_vmem, sem):
    @pl.when(pl.program_id(0) == 0)
    def _():
        cp = pltpu.make_async_copy(w_hbm, w_vmem, sem)
        cp.start()
        cp.wait()

    o_ref[...] = (
        jnp.dot(s_ref[...], w_vmem[...], preferred_element_type=jnp.float32)
        + bd_ref[...]
    )


@jax.jit
def kernel(x, W_enc, b_enc, b_dec):
    n = x.shape[0]

    sparse = pl.pallas_call(
        _encode_topk_kernel,
        grid=(n // TB,),
        in_specs=[
            pl.BlockSpec((TB, D_IN), lambda i: (i, 0)),
            pl.BlockSpec(memory_space=pl.ANY),
            pl.BlockSpec((1, D_HIDDEN), lambda i: (0, 0)),
        ],
        out_specs=pl.BlockSpec((TB, D_HIDDEN), lambda i: (i, 0)),
        out_shape=jax.ShapeDtypeStruct((n, D_HIDDEN), jnp.float32),
        scratch_shapes=[
            pltpu.VMEM((D_HIDDEN, D_IN), jnp.float32),
            pltpu.SemaphoreType.DMA,
        ],
    )(x, W_enc, b_enc.reshape(1, D_HIDDEN))

    out = pl.pallas_call(
        _decode_kernel,
        grid=(n // TB_DEC,),
        in_specs=[
            pl.BlockSpec((TB_DEC, D_HIDDEN), lambda i: (i, 0)),
            pl.BlockSpec(memory_space=pl.ANY),
            pl.BlockSpec((1, D_IN), lambda i: (0, 0)),
        ],
        out_specs=pl.BlockSpec((TB_DEC, D_IN), lambda i: (i, 0)),
        out_shape=jax.ShapeDtypeStruct((n, D_IN), jnp.float32),
        scratch_shapes=[
            pltpu.VMEM((D_HIDDEN, D_IN), jnp.float32),
            pltpu.SemaphoreType.DMA,
        ],
    )(sparse, W_enc, b_dec.reshape(1, D_IN))

    return out
